# Initial kernel scaffold; baseline (speedup 1.0000x reference)
import jax
import jax.numpy as jnp
from jax.experimental import pallas as pl


def kernel(x, W, b):
    raise NotImplementedError("write your pallas kernel here")


# speedup vs baseline: 5.9241x; 5.9241x over previous
import functools

import jax
import jax.numpy as jnp
from jax import lax
from jax.experimental import pallas as pl
from jax.experimental.pallas import tpu as pltpu
from jax.experimental.pallas import tpu_sc as plsc

N = 16384
F = 128
K = 819
BN = 2048
NW = 32
COLS_PER_W = F // NW
L = 16
UNROLL = 8


def _logits_t_body(x_ref, w_ref, b_ref, out_ref):
    out_ref[...] = lax.dot_general(
        w_ref[...], x_ref[...], (((1,), (1,)), ((), ())),
        preferred_element_type=jnp.float32) + b_ref[...]


def _logits_t(x, W, b_col):
    return pl.pallas_call(
        _logits_t_body,
        grid=(N // BN,),
        in_specs=[
            pl.BlockSpec((BN, F), lambda i: (i, 0)),
            pl.BlockSpec((F, F), lambda i: (0, 0)),
            pl.BlockSpec((F, 1), lambda i: (0, 0)),
        ],
        out_specs=pl.BlockSpec((F, BN), lambda i: (0, i)),
        out_shape=jax.ShapeDtypeStruct((F, N), jnp.float32),
    )(x, W, b_col)


_mesh = plsc.VectorSubcoreMesh(core_axis_name="c", subcore_axis_name="s")


@functools.partial(
    pl.kernel,
    out_type=jax.ShapeDtypeStruct((F, L), jnp.int32),
    mesh=_mesh,
    scratch_types=[
        pltpu.VMEM((N,), jnp.float32),
        pltpu.VMEM((256 * L,), jnp.int32),
        pltpu.VMEM((L,), jnp.int32),
    ],
    compiler_params=pltpu.CompilerParams(needs_layout_passes=False),
)
def _select_kernel(logits_t, out_keys, col_v, hist_v, key_v):
    wid = lax.axis_index("s") * 2 + lax.axis_index("c")
    lanes = lax.iota(jnp.int32, L)
    ones = jnp.ones((L,), jnp.int32)
    zeros16 = jnp.zeros((L,), jnp.int32)

    for c in range(COLS_PER_W):
        col = wid * COLS_PER_W + c
        pltpu.sync_copy(logits_t.at[col], col_v)

        krem = jnp.int32(K)
        prefix = jnp.uint32(0)
        for level in range(4):
            shift = 24 - 8 * level

            def zbody(d, _):
                hist_v[pl.ds(d * L, L)] = zeros16
                return 0
            lax.fori_loop(0, 256, zbody, 0, unroll=8)

            def hbody(i, _, level=level, shift=shift, prefix=prefix):
                for u in range(UNROLL):
                    off = (i * UNROLL + u) * L
                    v = col_v[pl.ds(off, L)]
                    y = lax.bitcast_convert_type(v, jnp.uint32)
                    neg = y >= jnp.uint32(0x80000000)
                    key = jnp.where(neg, ~y, y | jnp.uint32(0x80000000))
                    digit = (key >> jnp.uint32(shift)) & jnp.uint32(0xFF)
                    idx = digit.astype(jnp.int32) * L + lanes
                    if level == 0:
                        plsc.addupdate_scatter(hist_v, [idx], ones)
                    else:
                        m = (key >> jnp.uint32(shift + 8)) == prefix
                        plsc.addupdate_scatter(hist_v, [idx], ones, mask=m)
                return 0
            lax.fori_loop(0, N // L // UNROLL, hbody, 0)

            def sbody(j, carry):
                cum, chosen, knext = carry
                d = 255 - j
                cnt = jnp.sum(hist_v[pl.ds(d * L, L)])
                newcum = cum + cnt
                hit = (cum < krem) & (newcum >= krem)
                chosen = jnp.where(hit, d, chosen)
                knext = jnp.where(hit, krem - cum, knext)
                return newcum, chosen, knext
            _, chosen, knext = lax.fori_loop(
                0, 256, sbody, (jnp.int32(0), jnp.int32(0), jnp.int32(1)))
            prefix = (prefix << jnp.uint32(8)) | chosen.astype(jnp.uint32)
            krem = knext

        key_v[...] = jnp.broadcast_to(
            lax.bitcast_convert_type(prefix, jnp.int32), (L,))
        pltpu.sync_copy(key_v, out_keys.at[col])


def _mask_body(x_ref, w_ref, b_ref, t_ref, out_ref):
    logits = lax.dot_general(
        x_ref[...], w_ref[...], (((1,), (1,)), ((), ())),
        preferred_element_type=jnp.float32) + b_ref[...]
    out_ref[...] = (logits >= t_ref[...]).astype(jnp.float32)


def _mask(x, W, b_row, thr_row):
    return pl.pallas_call(
        _mask_body,
        grid=(N // BN,),
        in_specs=[
            pl.BlockSpec((BN, F), lambda i: (i, 0)),
            pl.BlockSpec((F, F), lambda i: (0, 0)),
            pl.BlockSpec((1, F), lambda i: (0, 0)),
            pl.BlockSpec((1, F), lambda i: (0, 0)),
        ],
        out_specs=pl.BlockSpec((BN, F), lambda i: (i, 0)),
        out_shape=jax.ShapeDtypeStruct((N, F), jnp.float32),
    )(x, W, b_row, thr_row)


def kernel(x, W, b):
    logits_t = _logits_t(x, W, b.reshape(F, 1))
    keys = _select_kernel(logits_t)
    ku = lax.bitcast_convert_type(keys[:, 0], jnp.uint32)
    orig = jnp.where(ku >= jnp.uint32(0x80000000),
                     ku ^ jnp.uint32(0x80000000), ~ku)
    thr = lax.bitcast_convert_type(orig, jnp.float32).reshape(1, F)
    return _mask(x, W, b.reshape(1, F), thr)


# speedup vs baseline: 6.7177x; 1.1340x over previous
import functools

import jax
import jax.numpy as jnp
from jax import lax
from jax.experimental import pallas as pl
from jax.experimental.pallas import tpu as pltpu
from jax.experimental.pallas import tpu_sc as plsc

N = 16384
F = 128
K = 819
BN = 2048
NW = 32
COLS_PER_W = F // NW
L = 16
UNROLL = 8


def _logits_t_body(x_ref, w_ref, b_ref, out_ref):
    out_ref[...] = lax.dot_general(
        w_ref[...], x_ref[...], (((1,), (1,)), ((), ())),
        preferred_element_type=jnp.float32) + b_ref[...]


def _logits_t(x, W, b_col):
    return pl.pallas_call(
        _logits_t_body,
        grid=(N // BN,),
        in_specs=[
            pl.BlockSpec((BN, F), lambda i: (i, 0)),
            pl.BlockSpec((F, F), lambda i: (0, 0)),
            pl.BlockSpec((F, 1), lambda i: (0, 0)),
        ],
        out_specs=pl.BlockSpec((F, BN), lambda i: (0, i)),
        out_shape=jax.ShapeDtypeStruct((F, N), jnp.float32),
    )(x, W, b_col)


_mesh = plsc.VectorSubcoreMesh(core_axis_name="c", subcore_axis_name="s")


@functools.partial(
    pl.kernel,
    out_type=jax.ShapeDtypeStruct((F, L), jnp.int32),
    mesh=_mesh,
    scratch_types=[
        pltpu.VMEM((N,), jnp.float32),
        pltpu.VMEM((N,), jnp.int32),
        pltpu.VMEM((256 * L,), jnp.int32),
        pltpu.VMEM((L,), jnp.int32),
    ],
    compiler_params=pltpu.CompilerParams(needs_layout_passes=False),
)
def _select_kernel(logits_t, out_keys, col_v, key_v, hist_v, stage_v):
    wid = lax.axis_index("s") * 2 + lax.axis_index("c")
    lanes = lax.iota(jnp.int32, L)
    ones = jnp.ones((L,), jnp.int32)
    zeros16 = jnp.zeros((L,), jnp.int32)

    def digit_search(krem):
        def gbody(j, carry):
            cum, gch, kg = carry
            g = 15 - j
            acc = hist_v[pl.ds(g * 256, L)]
            for t in range(1, 16):
                acc = acc + hist_v[pl.ds(g * 256 + t * L, L)]
            cnt = jnp.sum(acc)
            newcum = cum + cnt
            hit = (cum < krem) & (newcum >= krem)
            gch = jnp.where(hit, g, gch)
            kg = jnp.where(hit, krem - cum, kg)
            return newcum, gch, kg
        _, gch, kg = lax.fori_loop(
            0, 16, gbody, (jnp.int32(0), jnp.int32(0), jnp.int32(1)))

        def dbody(j, carry):
            cum, dch, kn = carry
            t = 15 - j
            cnt = jnp.sum(hist_v[pl.ds(gch * 256 + t * L, L)])
            newcum = cum + cnt
            hit = (cum < kg) & (newcum >= kg)
            dch = jnp.where(hit, t, dch)
            kn = jnp.where(hit, kg - cum, kn)
            return newcum, dch, kn
        _, dch, kn = lax.fori_loop(
            0, 16, dbody, (jnp.int32(0), jnp.int32(0), jnp.int32(1)))
        return gch * 16 + dch, kn

    for c in range(COLS_PER_W):
        col = wid * COLS_PER_W + c
        pltpu.sync_copy(logits_t.at[col], col_v)

        def zbody(d, _):
            hist_v[pl.ds(d * L, L)] = zeros16
            return 0
        lax.fori_loop(0, 256, zbody, 0, unroll=8)

        def hbody0(i, _):
            for u in range(UNROLL):
                off = (i * UNROLL + u) * L
                v = col_v[pl.ds(off, L)]
                y = lax.bitcast_convert_type(v, jnp.uint32)
                neg = y >= jnp.uint32(0x80000000)
                key = jnp.where(neg, ~y, y | jnp.uint32(0x80000000))
                key_v[pl.ds(off, L)] = lax.bitcast_convert_type(key, jnp.int32)
                idx = (key >> jnp.uint32(24)).astype(jnp.int32) * L + lanes
                plsc.addupdate_scatter(hist_v, [idx], ones)
            return 0
        lax.fori_loop(0, N // L // UNROLL, hbody0, 0)

        chosen, krem = digit_search(jnp.int32(K))
        prefix = chosen.astype(jnp.uint32)

        for level in range(1, 4):
            shift = 24 - 8 * level
            lax.fori_loop(0, 256, zbody, 0, unroll=8)

            def hbody(i, _, shift=shift, prefix=prefix):
                for u in range(UNROLL):
                    off = (i * UNROLL + u) * L
                    key = lax.bitcast_convert_type(
                        key_v[pl.ds(off, L)], jnp.uint32)
                    m = (key >> jnp.uint32(shift + 8)) == prefix
                    digit = (key >> jnp.uint32(shift)) & jnp.uint32(0xFF)
                    idx = digit.astype(jnp.int32) * L + lanes
                    plsc.addupdate_scatter(hist_v, [idx], ones, mask=m)
                return 0
            lax.fori_loop(0, N // L // UNROLL, hbody, 0)

            chosen, krem = digit_search(krem)
            prefix = (prefix << jnp.uint32(8)) | chosen.astype(jnp.uint32)

        stage_v[...] = jnp.broadcast_to(
            lax.bitcast_convert_type(prefix, jnp.int32), (L,))
        pltpu.sync_copy(stage_v, out_keys.at[col])


def _mask_body(x_ref, w_ref, b_ref, t_ref, out_ref):
    logits = lax.dot_general(
        x_ref[...], w_ref[...], (((1,), (1,)), ((), ())),
        preferred_element_type=jnp.float32) + b_ref[...]
    out_ref[...] = (logits >= t_ref[...]).astype(jnp.float32)


def _mask(x, W, b_row, thr_row):
    return pl.pallas_call(
        _mask_body,
        grid=(N // BN,),
        in_specs=[
            pl.BlockSpec((BN, F), lambda i: (i, 0)),
            pl.BlockSpec((F, F), lambda i: (0, 0)),
            pl.BlockSpec((1, F), lambda i: (0, 0)),
            pl.BlockSpec((1, F), lambda i: (0, 0)),
        ],
        out_specs=pl.BlockSpec((BN, F), lambda i: (i, 0)),
        out_shape=jax.ShapeDtypeStruct((N, F), jnp.float32),
    )(x, W, b_row, thr_row)


def kernel(x, W, b):
    logits_t = _logits_t(x, W, b.reshape(F, 1))
    keys = _select_kernel(logits_t)
    ku = lax.bitcast_convert_type(keys[:, 0], jnp.uint32)
    orig = jnp.where(ku >= jnp.uint32(0x80000000),
                     ku ^ jnp.uint32(0x80000000), ~ku)
    thr = lax.bitcast_convert_type(orig, jnp.float32).reshape(1, F)
    return _mask(x, W, b.reshape(1, F), thr)


# speedup vs baseline: 16.2687x; 2.4218x over previous
import functools

import jax
import jax.numpy as jnp
from jax import lax
from jax.experimental import pallas as pl
from jax.experimental.pallas import tpu as pltpu
from jax.experimental.pallas import tpu_sc as plsc

N = 16384
F = 128
K = 819
BN = 2048
NW = 32
COLS_PER_W = F // NW
L = 16
UNROLL = 8


def _logits_t_body(x_ref, w_ref, b_ref, out_ref):
    out_ref[...] = lax.dot_general(
        w_ref[...], x_ref[...], (((1,), (1,)), ((), ())),
        preferred_element_type=jnp.float32) + b_ref[...]


def _logits_t(x, W, b_col):
    return pl.pallas_call(
        _logits_t_body,
        grid=(N // BN,),
        in_specs=[
            pl.BlockSpec((BN, F), lambda i: (i, 0)),
            pl.BlockSpec((F, F), lambda i: (0, 0)),
            pl.BlockSpec((F, 1), lambda i: (0, 0)),
        ],
        out_specs=pl.BlockSpec((F, BN), lambda i: (0, i)),
        out_shape=jax.ShapeDtypeStruct((F, N), jnp.float32),
    )(x, W, b_col)


_mesh = plsc.VectorSubcoreMesh(core_axis_name="c", subcore_axis_name="s")


@functools.partial(
    pl.kernel,
    out_type=jax.ShapeDtypeStruct((F, L), jnp.int32),
    mesh=_mesh,
    scratch_types=[
        pltpu.VMEM((N,), jnp.float32),
        pltpu.VMEM((N,), jnp.int32),
        pltpu.VMEM((256 * L,), jnp.int32),
        pltpu.VMEM((L,), jnp.int32),
    ],
    compiler_params=pltpu.CompilerParams(needs_layout_passes=False),
)
def _select_kernel(logits_t, out_keys, col_v, key_v, hist_v, stage_v):
    wid = lax.axis_index("s") * 2 + lax.axis_index("c")
    lanes = lax.iota(jnp.int32, L)
    ones = jnp.ones((L,), jnp.int32)
    zeros16 = jnp.zeros((L,), jnp.int32)

    def digit_search(krem):
        def gbody(j, carry):
            cum, gch, kg = carry
            g = 15 - j
            acc = hist_v[pl.ds(g * 256, L)]
            for t in range(1, 16):
                acc = acc + hist_v[pl.ds(g * 256 + t * L, L)]
            cnt = jnp.sum(acc)
            newcum = cum + cnt
            hit = (cum < krem) & (newcum >= krem)
            gch = jnp.where(hit, g, gch)
            kg = jnp.where(hit, krem - cum, kg)
            return newcum, gch, kg
        _, gch, kg = plsc.parallel_loop(
            0, 16, carry=(jnp.int32(0), jnp.int32(0), jnp.int32(1)))(gbody)

        def dbody(j, carry):
            cum, dch, kn = carry
            t = 15 - j
            cnt = jnp.sum(hist_v[pl.ds(gch * 256 + t * L, L)])
            newcum = cum + cnt
            hit = (cum < kg) & (newcum >= kg)
            dch = jnp.where(hit, t, dch)
            kn = jnp.where(hit, kg - cum, kn)
            return newcum, dch, kn
        _, dch, kn = plsc.parallel_loop(
            0, 16, carry=(jnp.int32(0), jnp.int32(0), jnp.int32(1)))(dbody)
        return gch * 16 + dch, kn

    for c in range(COLS_PER_W):
        col = wid * COLS_PER_W + c
        pltpu.sync_copy(logits_t.at[col], col_v)

        @plsc.parallel_loop(0, 256 * L, step=L, unroll=8)
        def _zero0(i):
            hist_v[pl.ds(i, L)] = zeros16

        @plsc.parallel_loop(0, N, step=L, unroll=UNROLL)
        def _hist0(i):
            v = col_v[pl.ds(i, L)]
            y = lax.bitcast_convert_type(v, jnp.uint32)
            neg = y >= jnp.uint32(0x80000000)
            key = jnp.where(neg, ~y, y | jnp.uint32(0x80000000))
            key_v[pl.ds(i, L)] = lax.bitcast_convert_type(key, jnp.int32)
            idx = ((key >> jnp.uint32(20)).astype(jnp.int32) & 0xFF0) + lanes
            plsc.addupdate_scatter(hist_v, [idx], ones)

        chosen, krem = digit_search(jnp.int32(K))
        prefix = chosen.astype(jnp.uint32)

        for level in range(1, 4):
            shift = 24 - 8 * level

            @plsc.parallel_loop(0, 256 * L, step=L, unroll=8)
            def _zero(i):
                hist_v[pl.ds(i, L)] = zeros16

            @plsc.parallel_loop(0, N, step=L, unroll=UNROLL)
            def _hist(i, shift=shift, prefix=prefix):
                key = lax.bitcast_convert_type(key_v[pl.ds(i, L)], jnp.uint32)
                m = (key >> jnp.uint32(shift + 8)) == prefix
                digit = (key >> jnp.uint32(shift)) & jnp.uint32(0xFF)
                idx = digit.astype(jnp.int32) * L + lanes
                plsc.addupdate_scatter(hist_v, [idx], ones, mask=m)

            chosen, krem = digit_search(krem)
            prefix = (prefix << jnp.uint32(8)) | chosen.astype(jnp.uint32)

        stage_v[...] = jnp.broadcast_to(
            lax.bitcast_convert_type(prefix, jnp.int32), (L,))
        pltpu.sync_copy(stage_v, out_keys.at[col])


def _mask_body(x_ref, w_ref, b_ref, t_ref, out_ref):
    logits = lax.dot_general(
        x_ref[...], w_ref[...], (((1,), (1,)), ((), ())),
        preferred_element_type=jnp.float32) + b_ref[...]
    out_ref[...] = (logits >= t_ref[...]).astype(jnp.float32)


def _mask(x, W, b_row, thr_row):
    return pl.pallas_call(
        _mask_body,
        grid=(N // BN,),
        in_specs=[
            pl.BlockSpec((BN, F), lambda i: (i, 0)),
            pl.BlockSpec((F, F), lambda i: (0, 0)),
            pl.BlockSpec((1, F), lambda i: (0, 0)),
            pl.BlockSpec((1, F), lambda i: (0, 0)),
        ],
        out_specs=pl.BlockSpec((BN, F), lambda i: (i, 0)),
        out_shape=jax.ShapeDtypeStruct((N, F), jnp.float32),
    )(x, W, b_row, thr_row)


def kernel(x, W, b):
    logits_t = _logits_t(x, W, b.reshape(F, 1))
    keys = _select_kernel(logits_t)
    ku = lax.bitcast_convert_type(keys[:, 0], jnp.uint32)
    orig = jnp.where(ku >= jnp.uint32(0x80000000),
                     ku ^ jnp.uint32(0x80000000), ~ku)
    thr = lax.bitcast_convert_type(orig, jnp.float32).reshape(1, F)
    return _mask(x, W, b.reshape(1, F), thr)


# speedup vs baseline: 16.5838x; 1.0194x over previous
import functools

import jax
import jax.numpy as jnp
from jax import lax
from jax.experimental import pallas as pl
from jax.experimental.pallas import tpu as pltpu
from jax.experimental.pallas import tpu_sc as plsc

N = 16384
F = 128
K = 819
BN = 2048
NW = 32
COLS_PER_W = F // NW
L = 16
UNROLL = 8


def _logits_t_body(x_ref, w_ref, b_ref, out_ref):
    out_ref[...] = lax.dot_general(
        w_ref[...], x_ref[...], (((1,), (1,)), ((), ())),
        preferred_element_type=jnp.float32) + b_ref[...]


def _logits_t(x, W, b_col):
    return pl.pallas_call(
        _logits_t_body,
        grid=(N // BN,),
        in_specs=[
            pl.BlockSpec((BN, F), lambda i: (i, 0)),
            pl.BlockSpec((F, F), lambda i: (0, 0)),
            pl.BlockSpec((F, 1), lambda i: (0, 0)),
        ],
        out_specs=pl.BlockSpec((F, BN), lambda i: (0, i)),
        out_shape=jax.ShapeDtypeStruct((F, N), jnp.float32),
    )(x, W, b_col)


_mesh = plsc.VectorSubcoreMesh(core_axis_name="c", subcore_axis_name="s")


@functools.partial(
    pl.kernel,
    out_type=jax.ShapeDtypeStruct((F, L), jnp.int32),
    mesh=_mesh,
    scratch_types=[
        pltpu.VMEM((N,), jnp.float32),
        pltpu.VMEM((N + L,), jnp.int32),
        pltpu.VMEM((N + L,), jnp.int32),
        pltpu.VMEM((256 * L,), jnp.int32),
        pltpu.VMEM((L,), jnp.int32),
    ],
    compiler_params=pltpu.CompilerParams(needs_layout_passes=False),
)
def _select_kernel(logits_t, out_keys, col_v, cand_a, cand_b, hist_v, stage_v):
    wid = lax.axis_index("s") * 2 + lax.axis_index("c")
    lanes = lax.iota(jnp.int32, L)
    ones = jnp.ones((L,), jnp.int32)
    zeros16 = jnp.zeros((L,), jnp.int32)

    def mono_key(v):
        y = lax.bitcast_convert_type(v, jnp.uint32)
        neg = y >= jnp.uint32(0x80000000)
        return jnp.where(neg, ~y, y | jnp.uint32(0x80000000))

    def digit_search(krem):
        def gbody(j, carry):
            cum, gch, kg = carry
            g = 15 - j
            acc = hist_v[pl.ds(g * 256, L)]
            for t in range(1, 16):
                acc = acc + hist_v[pl.ds(g * 256 + t * L, L)]
            cnt = jnp.sum(acc)
            newcum = cum + cnt
            hit = (cum < krem) & (newcum >= krem)
            gch = jnp.where(hit, g, gch)
            kg = jnp.where(hit, krem - cum, kg)
            return newcum, gch, kg
        _, gch, kg = plsc.parallel_loop(
            0, 16, carry=(jnp.int32(0), jnp.int32(0), jnp.int32(1)))(gbody)

        def dbody(j, carry):
            cum, dch, kn = carry
            t = 15 - j
            cnt = jnp.sum(hist_v[pl.ds(gch * 256 + t * L, L)])
            newcum = cum + cnt
            hit = (cum < kg) & (newcum >= kg)
            dch = jnp.where(hit, t, dch)
            kn = jnp.where(hit, kg - cum, kn)
            return newcum, dch, kn
        _, dch, kn = plsc.parallel_loop(
            0, 16, carry=(jnp.int32(0), jnp.int32(0), jnp.int32(1)))(dbody)
        return gch * 16 + dch, kn

    def zero_hist():
        @plsc.parallel_loop(0, 256 * L, step=L, unroll=8)
        def _z(i):
            hist_v[pl.ds(i, L)] = zeros16

    for c in range(COLS_PER_W):
        col = wid * COLS_PER_W + c
        pltpu.sync_copy(logits_t.at[col], col_v)

        zero_hist()

        @plsc.parallel_loop(0, N, step=L, unroll=UNROLL)
        def _hist0(i):
            key = mono_key(col_v[pl.ds(i, L)])
            idx = ((key >> jnp.uint32(20)).astype(jnp.int32) & 0xFF0) + lanes
            plsc.addupdate_scatter(hist_v, [idx], ones)

        chosen, krem = digit_search(jnp.int32(K))
        prefix = chosen.astype(jnp.uint32)

        def fbody0(i, ptr):
            key = mono_key(col_v[pl.ds(i, L)])
            m = (key >> jnp.uint32(24)) == prefix
            addr = ptr + plsc.cumsum(m.astype(jnp.int32)) - 1
            plsc.store_scatter(cand_a, [addr],
                               lax.bitcast_convert_type(key, jnp.int32),
                               mask=m)
            return ptr + plsc.all_reduce_population_count(m)
        cnt_vec = plsc.parallel_loop(
            0, N, step=L, unroll=UNROLL,
            carry=jnp.zeros((L,), jnp.int32))(fbody0)

        src, dst = cand_a, cand_b
        for level in range(1, 4):
            shift = 24 - 8 * level
            stage_v[...] = cnt_vec
            cnt = stage_v[...][0]
            base = (cnt // L) * L
            rem = cnt - base

            zero_hist()

            @plsc.parallel_loop(0, base, step=L, unroll=4)
            def _hist(i, shift=shift, src=src):
                key = lax.bitcast_convert_type(src[pl.ds(i, L)], jnp.uint32)
                digit = (key >> jnp.uint32(shift)) & jnp.uint32(0xFF)
                idx = digit.astype(jnp.int32) * L + lanes
                plsc.addupdate_scatter(hist_v, [idx], ones)
            tail_m = lanes < rem
            key_t = lax.bitcast_convert_type(src[pl.ds(base, L)], jnp.uint32)
            digit_t = (key_t >> jnp.uint32(shift)) & jnp.uint32(0xFF)
            plsc.addupdate_scatter(
                hist_v, [digit_t.astype(jnp.int32) * L + lanes], ones,
                mask=tail_m)

            chosen, krem = digit_search(krem)
            prefix = (prefix << jnp.uint32(8)) | chosen.astype(jnp.uint32)

            if level < 3:
                def fbody(i, ptr, shift=shift, src=src, dst=dst,
                          prefix=prefix):
                    key = lax.bitcast_convert_type(
                        src[pl.ds(i, L)], jnp.uint32)
                    m = (key >> jnp.uint32(shift)) == prefix
                    addr = ptr + plsc.cumsum(m.astype(jnp.int32)) - 1
                    plsc.store_scatter(
                        dst, [addr],
                        lax.bitcast_convert_type(key, jnp.int32), mask=m)
                    return ptr + plsc.all_reduce_population_count(m)
                ptr = plsc.parallel_loop(
                    0, base, step=L, unroll=4,
                    carry=jnp.zeros((L,), jnp.int32))(fbody)
                m_t = ((key_t >> jnp.uint32(shift)) == prefix) & tail_m
                addr_t = ptr + plsc.cumsum(m_t.astype(jnp.int32)) - 1
                plsc.store_scatter(
                    dst, [addr_t],
                    lax.bitcast_convert_type(key_t, jnp.int32), mask=m_t)
                cnt_vec = ptr + plsc.all_reduce_population_count(m_t)
                src, dst = dst, src

        stage_v[...] = jnp.broadcast_to(
            lax.bitcast_convert_type(prefix, jnp.int32), (L,))
        pltpu.sync_copy(stage_v, out_keys.at[col])


def _mask_body(x_ref, w_ref, b_ref, t_ref, out_ref):
    logits = lax.dot_general(
        x_ref[...], w_ref[...], (((1,), (1,)), ((), ())),
        preferred_element_type=jnp.float32) + b_ref[...]
    out_ref[...] = (logits >= t_ref[...]).astype(jnp.float32)


def _mask(x, W, b_row, thr_row):
    return pl.pallas_call(
        _mask_body,
        grid=(N // BN,),
        in_specs=[
            pl.BlockSpec((BN, F), lambda i: (i, 0)),
            pl.BlockSpec((F, F), lambda i: (0, 0)),
            pl.BlockSpec((1, F), lambda i: (0, 0)),
            pl.BlockSpec((1, F), lambda i: (0, 0)),
        ],
        out_specs=pl.BlockSpec((BN, F), lambda i: (i, 0)),
        out_shape=jax.ShapeDtypeStruct((N, F), jnp.float32),
    )(x, W, b_row, thr_row)


def kernel(x, W, b):
    logits_t = _logits_t(x, W, b.reshape(F, 1))
    keys = _select_kernel(logits_t)
    ku = lax.bitcast_convert_type(keys[:, 0], jnp.uint32)
    orig = jnp.where(ku >= jnp.uint32(0x80000000),
                     ku ^ jnp.uint32(0x80000000), ~ku)
    thr = lax.bitcast_convert_type(orig, jnp.float32).reshape(1, F)
    return _mask(x, W, b.reshape(1, F), thr)


# speedup vs baseline: 19.0060x; 1.1461x over previous
import functools

import jax
import jax.numpy as jnp
from jax import lax
from jax.experimental import pallas as pl
from jax.experimental.pallas import tpu as pltpu
from jax.experimental.pallas import tpu_sc as plsc

N = 16384
F = 128
K = 819
BN = 2048
NW = 32
COLS_PER_W = F // NW
L = 16
UNROLL = 8


def _logits_t_body(x_ref, w_ref, out_ref):
    out_ref[...] = lax.dot_general(
        w_ref[...], x_ref[...], (((1,), (1,)), ((), ())),
        preferred_element_type=jnp.float32)


def _logits_t(x, W):
    return pl.pallas_call(
        _logits_t_body,
        grid=(N // BN,),
        in_specs=[
            pl.BlockSpec((BN, F), lambda i: (i, 0)),
            pl.BlockSpec((F, F), lambda i: (0, 0)),
        ],
        out_specs=pl.BlockSpec((F, BN), lambda i: (0, i)),
        out_shape=jax.ShapeDtypeStruct((F, N), jnp.float32),
    )(x, W)


_mesh = plsc.VectorSubcoreMesh(core_axis_name="c", subcore_axis_name="s")


@functools.partial(
    pl.kernel,
    out_type=jax.ShapeDtypeStruct((F, 128), jnp.int32),
    mesh=_mesh,
    scratch_types=[
        pltpu.VMEM((N,), jnp.float32),
        pltpu.VMEM((N,), jnp.float32),
        pltpu.VMEM((N + L,), jnp.int32),
        pltpu.VMEM((N + L,), jnp.int32),
        pltpu.VMEM((256 * L,), jnp.int32),
        pltpu.VMEM((L,), jnp.int32),
        pltpu.VMEM((128,), jnp.int32),
        pltpu.SemaphoreType.DMA,
        pltpu.SemaphoreType.DMA,
    ],
    compiler_params=pltpu.CompilerParams(needs_layout_passes=False),
)
def _select_kernel(logits_t, out_keys, col_a2, col_b2, cand_a, cand_b,
                   hist_v, stage_v, stage128_v, sem_a, sem_b):
    wid = lax.axis_index("s") * 2 + lax.axis_index("c")
    lanes = lax.iota(jnp.int32, L)
    ones = jnp.ones((L,), jnp.int32)
    zeros16 = jnp.zeros((L,), jnp.int32)

    def digit_search(krem, first, neg):
        def gbody(j, carry):
            cum, gch, kg = carry
            if first:
                g = jnp.where(j < 8, 7 - j, j)
            else:
                g = jnp.where(neg, j, 15 - j)
            acc = hist_v[pl.ds(g * 256, L)]
            for t in range(1, 16):
                acc = acc + hist_v[pl.ds(g * 256 + t * L, L)]
            cnt = jnp.sum(acc)
            newcum = cum + cnt
            hit = (cum < krem) & (newcum >= krem)
            gch = jnp.where(hit, g, gch)
            kg = jnp.where(hit, krem - cum, kg)
            return newcum, gch, kg
        _, gch, kg = plsc.parallel_loop(
            0, 16, carry=(jnp.int32(0), jnp.int32(0), jnp.int32(1)))(gbody)

        gneg = (gch >= 8) if first else neg

        def dbody(j, carry):
            cum, dch, kn = carry
            t = jnp.where(gneg, j, 15 - j)
            cnt = jnp.sum(hist_v[pl.ds(gch * 256 + t * L, L)])
            newcum = cum + cnt
            hit = (cum < kg) & (newcum >= kg)
            dch = jnp.where(hit, t, dch)
            kn = jnp.where(hit, kg - cum, kn)
            return newcum, dch, kn
        _, dch, kn = plsc.parallel_loop(
            0, 16, carry=(jnp.int32(0), jnp.int32(0), jnp.int32(1)))(dbody)
        return gch * 16 + dch, kn

    def zero_hist():
        @plsc.parallel_loop(0, 256 * L, step=L, unroll=8)
        def _z(i):
            hist_v[pl.ds(i, L)] = zeros16

    h = pltpu.async_copy(logits_t.at[wid * COLS_PER_W], col_a2, sem_a)
    for c in range(COLS_PER_W):
        col = wid * COLS_PER_W + c
        col_v = col_a2 if c % 2 == 0 else col_b2
        h.wait()
        if c + 1 < COLS_PER_W:
            nxt = col_b2 if c % 2 == 0 else col_a2
            nsem = sem_b if c % 2 == 0 else sem_a
            h = pltpu.async_copy(logits_t.at[col + 1], nxt, nsem)

        zero_hist()

        @plsc.parallel_loop(0, N, step=L, unroll=UNROLL)
        def _hist0(i, col_v=col_v):
            y = lax.bitcast_convert_type(col_v[pl.ds(i, L)], jnp.uint32)
            idx = ((y >> jnp.uint32(20)).astype(jnp.int32) & 0xFF0) + lanes
            plsc.addupdate_scatter(hist_v, [idx], ones)

        chosen, krem = digit_search(jnp.int32(K), True, False)
        prefix = chosen.astype(jnp.uint32)
        neg = chosen >= 128

        def fbody0(i, ptr, col_v=col_v):
            y = lax.bitcast_convert_type(col_v[pl.ds(i, L)], jnp.uint32)
            m = (y >> jnp.uint32(24)) == prefix
            addr = ptr + plsc.cumsum(m.astype(jnp.int32)) - 1
            plsc.store_scatter(cand_a, [addr],
                               lax.bitcast_convert_type(y, jnp.int32),
                               mask=m)
            return ptr + plsc.all_reduce_population_count(m)
        cnt_vec = plsc.parallel_loop(
            0, N, step=L, unroll=UNROLL,
            carry=jnp.zeros((L,), jnp.int32))(fbody0)

        src, dst = cand_a, cand_b
        for level in range(1, 4):
            shift = 24 - 8 * level
            stage_v[...] = cnt_vec
            cnt = stage_v[...][0]
            base = (cnt // L) * L
            rem = cnt - base

            zero_hist()

            @plsc.parallel_loop(0, base, step=L, unroll=4)
            def _hist(i, shift=shift, src=src):
                key = lax.bitcast_convert_type(src[pl.ds(i, L)], jnp.uint32)
                digit = (key >> jnp.uint32(shift)) & jnp.uint32(0xFF)
                idx = digit.astype(jnp.int32) * L + lanes
                plsc.addupdate_scatter(hist_v, [idx], ones)
            tail_m = lanes < rem
            key_t = lax.bitcast_convert_type(src[pl.ds(base, L)], jnp.uint32)
            digit_t = (key_t >> jnp.uint32(shift)) & jnp.uint32(0xFF)
            plsc.addupdate_scatter(
                hist_v, [digit_t.astype(jnp.int32) * L + lanes], ones,
                mask=tail_m)

            chosen, krem = digit_search(krem, False, neg)
            prefix = (prefix << jnp.uint32(8)) | chosen.astype(jnp.uint32)

            if level < 3:
                def fbody(i, ptr, shift=shift, src=src, dst=dst,
                          prefix=prefix):
                    key = lax.bitcast_convert_type(
                        src[pl.ds(i, L)], jnp.uint32)
                    m = (key >> jnp.uint32(shift)) == prefix
                    addr = ptr + plsc.cumsum(m.astype(jnp.int32)) - 1
                    plsc.store_scatter(
                        dst, [addr],
                        lax.bitcast_convert_type(key, jnp.int32), mask=m)
                    return ptr + plsc.all_reduce_population_count(m)
                ptr = plsc.parallel_loop(
                    0, base, step=L, unroll=4,
                    carry=jnp.zeros((L,), jnp.int32))(fbody)
                m_t = ((key_t >> jnp.uint32(shift)) == prefix) & tail_m
                addr_t = ptr + plsc.cumsum(m_t.astype(jnp.int32)) - 1
                plsc.store_scatter(
                    dst, [addr_t],
                    lax.bitcast_convert_type(key_t, jnp.int32), mask=m_t)
                cnt_vec = ptr + plsc.all_reduce_population_count(m_t)
                src, dst = dst, src

        pbc = jnp.broadcast_to(lax.bitcast_convert_type(prefix, jnp.int32),
                               (L,))

        @plsc.parallel_loop(0, 128, step=L)
        def _st(i, pbc=pbc):
            stage128_v[pl.ds(i, L)] = pbc
        pltpu.sync_copy(stage128_v, out_keys.at[col])


def _mask_body(x_ref, w_ref, t_ref, out_ref):
    logits = lax.dot_general(
        x_ref[...], w_ref[...], (((1,), (1,)), ((), ())),
        preferred_element_type=jnp.float32)
    thr2 = lax.bitcast_convert_type(t_ref[...], jnp.float32)
    thr_row = lax.transpose(thr2, (1, 0))[0:1, :]
    out_ref[...] = (logits >= thr_row).astype(jnp.float32)


def _mask(x, W, keys):
    return pl.pallas_call(
        _mask_body,
        grid=(N // BN,),
        in_specs=[
            pl.BlockSpec((BN, F), lambda i: (i, 0)),
            pl.BlockSpec((F, F), lambda i: (0, 0)),
            pl.BlockSpec((F, 128), lambda i: (0, 0)),
        ],
        out_specs=pl.BlockSpec((BN, F), lambda i: (i, 0)),
        out_shape=jax.ShapeDtypeStruct((N, F), jnp.float32),
    )(x, W, keys)


def kernel(x, W, b):
    del b
    logits_t = _logits_t(x, W)
    keys = _select_kernel(logits_t)
    return _mask(x, W, keys)
